# Initial kernel scaffold; baseline (speedup 1.0000x reference)
import jax
import jax.numpy as jnp
from jax.experimental import pallas as pl


def kernel(final_relation_representations, query_rels, similarity_threshold_raw, enhancement_strength_raw, similarity_weight_scale, temperature):
    raise NotImplementedError("write your pallas kernel here")


# speedup vs baseline: 1.1711x; 1.1711x over previous
import jax
import jax.numpy as jnp
from jax.experimental import pallas as pl
from jax.experimental.pallas import tpu as pltpu


def _enhance_block(x_ref, qr_ref, s_ref, out_ref):
    x = x_ref[...]
    qr = qr_ref[...]
    threshold = s_ref[0]
    strength = s_ref[1]
    scale = s_ref[2]
    inv_temp = s_ref[3]

    bB, R, D = x.shape
    ridx3 = jax.lax.broadcasted_iota(jnp.int32, (bB, R, D), 1)
    ridx1 = jax.lax.broadcasted_iota(jnp.int32, (bB, R, 1), 1)
    is_q3 = ridx3 == qr
    is_q1 = ridx1 == qr

    q = jnp.sum(jnp.where(is_q3, x, 0.0), axis=1, keepdims=True)
    qnorm = jnp.sqrt(jnp.sum(q * q, axis=2, keepdims=True))
    qn = q / jnp.maximum(qnorm, 1e-12)

    row_n = jnp.sqrt(jnp.sum(x * x, axis=2, keepdims=True))
    dots = jnp.sum(x * qn, axis=2, keepdims=True)
    sims = dots / jnp.maximum(row_n, 1e-12)
    sims = jnp.where(is_q1, -1.0, sims)

    sim_w = jax.nn.sigmoid((sims - threshold) * 10.0)
    maskf = jnp.where(sim_w > 0.5, 1.0, 0.0)
    expw = maskf * jnp.exp(sims * inv_temp)
    denom = jnp.sum(expw, axis=1, keepdims=True)
    weights = expw / jnp.where(denom > 0, denom, 1.0)
    combined = weights * sim_w
    adjusted = combined * (1.0 + scale * sims)
    adjusted = adjusted / (jnp.sum(adjusted, axis=1, keepdims=True) + 1e-08)

    ws = jnp.sum(adjusted * x, axis=1, keepdims=True)
    enhanced = (1.0 - strength) * q + strength * ws
    any_valid = jnp.sum(maskf, axis=1, keepdims=True) > 0.0
    final_q = jnp.where(any_valid, enhanced, q)

    out_ref[...] = jnp.where(is_q3, final_q, x)


def kernel(final_relation_representations, query_rels, similarity_threshold_raw,
           enhancement_strength_raw, similarity_weight_scale, temperature):
    reprs = final_relation_representations
    B, R, D = reprs.shape
    bB = 8

    threshold = jax.nn.sigmoid(similarity_threshold_raw)
    strength = jax.nn.sigmoid(enhancement_strength_raw) * 0.2
    temp = jnp.clip(temperature, 0.1, 10.0)
    scalars = jnp.stack([threshold, strength, similarity_weight_scale,
                         1.0 / temp]).astype(jnp.float32)

    qr3 = query_rels.astype(jnp.int32).reshape(B, 1, 1)

    out = pl.pallas_call(
        _enhance_block,
        grid=(B // bB,),
        in_specs=[
            pl.BlockSpec((bB, R, D), lambda i: (i, 0, 0)),
            pl.BlockSpec((bB, 1, 1), lambda i: (i, 0, 0)),
            pl.BlockSpec(memory_space=pltpu.SMEM),
        ],
        out_specs=pl.BlockSpec((bB, R, D), lambda i: (i, 0, 0)),
        out_shape=jax.ShapeDtypeStruct((B, R, D), jnp.float32),
    )(reprs, qr3, scalars)
    return out


# speedup vs baseline: 1.5986x; 1.3650x over previous
import jax
import jax.numpy as jnp
from jax.experimental import pallas as pl
from jax.experimental.pallas import tpu as pltpu


def _enhance_block(qr_smem, x_ref, s_ref, out_ref):
    bB, R, D = x_ref.shape
    threshold = s_ref[0]
    strength = s_ref[1]
    scale = s_ref[2]
    inv_temp = s_ref[3]
    i = pl.program_id(0)

    x = x_ref[...]
    xsq = x * x
    out_ref[...] = x

    ridx = jax.lax.broadcasted_iota(jnp.int32, (1, R), 1)

    for b in range(bB):
        qr = qr_smem[i * bB + b]
        X = x_ref[b]
        q = x_ref[b, pl.ds(qr, 1), :]
        qn = q / jnp.maximum(
            jnp.sqrt(jnp.sum(q * q, axis=1, keepdims=True)), 1e-12)

        dn = (((1,), (1,)), ((), ()))
        dots = jax.lax.dot_general(qn, X, dn,
                                   preferred_element_type=jnp.float32)
        row_sq = jax.lax.dot_general(jnp.ones((1, D), jnp.float32), xsq[b], dn,
                                     preferred_element_type=jnp.float32)
        sims = dots / jnp.maximum(jnp.sqrt(row_sq), 1e-12)
        sims = jnp.where(ridx == qr, -1.0, sims)

        sim_w = jax.nn.sigmoid((sims - threshold) * 10.0)
        maskf = jnp.where(sim_w > 0.5, 1.0, 0.0)
        expw = maskf * jnp.exp(sims * inv_temp)
        denom = jnp.sum(expw, axis=1, keepdims=True)
        weights = expw / jnp.where(denom > 0, denom, 1.0)
        combined = weights * sim_w
        adjusted = combined * (1.0 + scale * sims)
        adjusted = adjusted / (jnp.sum(adjusted, axis=1, keepdims=True) + 1e-08)

        ws = jnp.dot(adjusted, X, preferred_element_type=jnp.float32)
        enhanced = (1.0 - strength) * q + strength * ws
        any_valid = jnp.sum(maskf, axis=1, keepdims=True) > 0.0
        final_q = jnp.where(any_valid, enhanced, q)

        out_ref[b, pl.ds(qr, 1), :] = final_q


def kernel(final_relation_representations, query_rels, similarity_threshold_raw,
           enhancement_strength_raw, similarity_weight_scale, temperature):
    reprs = final_relation_representations
    B, R, D = reprs.shape
    bB = 8

    threshold = jax.nn.sigmoid(similarity_threshold_raw)
    strength = jax.nn.sigmoid(enhancement_strength_raw) * 0.2
    temp = jnp.clip(temperature, 0.1, 10.0)
    scalars = jnp.stack([threshold, strength, similarity_weight_scale,
                         1.0 / temp]).astype(jnp.float32)

    grid_spec = pltpu.PrefetchScalarGridSpec(
        num_scalar_prefetch=1,
        grid=(B // bB,),
        in_specs=[
            pl.BlockSpec((bB, R, D), lambda i, qr: (i, 0, 0)),
            pl.BlockSpec(memory_space=pltpu.SMEM),
        ],
        out_specs=pl.BlockSpec((bB, R, D), lambda i, qr: (i, 0, 0)),
    )

    out = pl.pallas_call(
        _enhance_block,
        grid_spec=grid_spec,
        out_shape=jax.ShapeDtypeStruct((B, R, D), jnp.float32),
    )(query_rels.astype(jnp.int32), reprs, scalars)
    return out


# speedup vs baseline: 1.8515x; 1.1582x over previous
import jax
import jax.numpy as jnp
from jax.experimental import pallas as pl
from jax.experimental.pallas import tpu as pltpu


def _enhance_block(qr_smem, x_ref, s_ref, out_ref):
    bB, R, D = x_ref.shape
    threshold = s_ref[0]
    strength = s_ref[1]
    scale = s_ref[2]
    inv_temp = s_ref[3]
    i = pl.program_id(0)

    x = x_ref[...]
    xsq = x * x
    out_ref[...] = x

    ridx = jax.lax.broadcasted_iota(jnp.int32, (1, R), 1)
    dn = (((1,), (1,)), ((), ()))
    ones_d = jnp.ones((1, D), jnp.float32)

    for b in range(bB):
        qr = qr_smem[i * bB + b]
        X = x_ref[b]
        q = x_ref[b, pl.ds(qr, 1), :]

        dots = jax.lax.dot_general(q, X, dn,
                                   preferred_element_type=jnp.float32)
        row_sq = jax.lax.dot_general(ones_d, xsq[b], dn,
                                     preferred_element_type=jnp.float32)
        inv_q = 1.0 / jnp.maximum(
            jnp.sqrt(jnp.sum(q * q, axis=1, keepdims=True)), 1e-12)
        sims = (dots * inv_q) / jnp.maximum(jnp.sqrt(row_sq), 1e-12)
        sims = jnp.where(ridx == qr, -1.0, sims)

        sim_w = jax.nn.sigmoid((sims - threshold) * 10.0)
        maskf = jnp.where(sim_w > 0.5, 1.0, 0.0)
        expw = maskf * jnp.exp(sims * inv_temp)
        u = expw * sim_w * (1.0 + scale * sims)

        denom = jnp.sum(expw, axis=1, keepdims=True)
        su = jnp.sum(u, axis=1, keepdims=True)
        anyv = jnp.sum(maskf, axis=1, keepdims=True)
        uX = jnp.dot(u, X, preferred_element_type=jnp.float32)

        denom1 = jnp.where(denom > 0, denom, 1.0)
        ws = uX * (1.0 / (su + 1e-08 * denom1))
        enhanced = (1.0 - strength) * q + strength * ws
        final_q = jnp.where(anyv > 0.0, enhanced, q)

        out_ref[b, pl.ds(qr, 1), :] = final_q


def kernel(final_relation_representations, query_rels, similarity_threshold_raw,
           enhancement_strength_raw, similarity_weight_scale, temperature):
    reprs = final_relation_representations
    B, R, D = reprs.shape
    bB = 8

    threshold = jax.nn.sigmoid(similarity_threshold_raw)
    strength = jax.nn.sigmoid(enhancement_strength_raw) * 0.2
    temp = jnp.clip(temperature, 0.1, 10.0)
    scalars = jnp.stack([threshold, strength, similarity_weight_scale,
                         1.0 / temp]).astype(jnp.float32)

    grid_spec = pltpu.PrefetchScalarGridSpec(
        num_scalar_prefetch=1,
        grid=(B // bB,),
        in_specs=[
            pl.BlockSpec((bB, R, D), lambda i, qr: (i, 0, 0)),
            pl.BlockSpec(memory_space=pltpu.SMEM),
        ],
        out_specs=pl.BlockSpec((bB, R, D), lambda i, qr: (i, 0, 0)),
    )

    out = pl.pallas_call(
        _enhance_block,
        grid_spec=grid_spec,
        out_shape=jax.ShapeDtypeStruct((B, R, D), jnp.float32),
    )(query_rels.astype(jnp.int32), reprs, scalars)
    return out
